# Initial kernel scaffold; baseline (speedup 1.0000x reference)
import jax
import jax.numpy as jnp
from jax.experimental import pallas as pl


def kernel(x, w1, w2):
    raise NotImplementedError("write your pallas kernel here")


# speedup vs baseline: 1.0055x; 1.0055x over previous
import functools

import jax
import jax.numpy as jnp
from jax.experimental import pallas as pl
from jax.experimental.pallas import tpu as pltpu

_VMEM_LIMIT = 100 * 1024 * 1024


def _fused_body(x_ref, w1t_ref, w2t_ref, o_ref, *, inv_hw):
    x = x_ref[...]
    xf = x.astype(jnp.float32)
    avg = jnp.sum(xf, axis=-1) * inv_hw
    mx = jnp.max(xf, axis=-1)
    w1t = w1t_ref[...]
    h = (jnp.maximum(jnp.dot(avg, w1t, preferred_element_type=jnp.float32), 0.0)
         + jnp.maximum(jnp.dot(mx, w1t, preferred_element_type=jnp.float32), 0.0))
    f = jnp.dot(h, w2t_ref[...], preferred_element_type=jnp.float32)
    attn = jax.nn.sigmoid(f)
    o_ref[...] = (xf * attn[:, :, None]).astype(o_ref.dtype)


def _run_fused(x_flat, w1t, w2t, bt):
    n, c, hw = x_flat.shape
    cr = w1t.shape[1]
    body = functools.partial(_fused_body, inv_hw=1.0 / float(hw))
    return pl.pallas_call(
        body,
        out_shape=jax.ShapeDtypeStruct((n, c, hw), x_flat.dtype),
        grid=(n // bt,),
        in_specs=[
            pl.BlockSpec((bt, c, hw), lambda b: (b, 0, 0)),
            pl.BlockSpec((c, cr), lambda b: (0, 0)),
            pl.BlockSpec((cr, c), lambda b: (0, 0)),
        ],
        out_specs=pl.BlockSpec((bt, c, hw), lambda b: (b, 0, 0)),
        compiler_params=pltpu.CompilerParams(
            dimension_semantics=("parallel",),
            vmem_limit_bytes=_VMEM_LIMIT,
        ),
    )(x_flat, w1t, w2t)


def kernel(x, w1, w2):
    n, c, h, w = x.shape
    cr = w1.shape[0]
    hw = h * w
    x_flat = x.reshape(n, c, hw)
    row_bytes = c * hw * jnp.dtype(x.dtype).itemsize

    budget = 8 * 1024 * 1024
    bt = 1
    for d in range(1, n + 1):
        if n % d == 0 and d * row_bytes <= budget and n // d >= 2:
            bt = d

    w1t = jnp.transpose(w1).astype(jnp.float32)
    w2t = jnp.transpose(w2).astype(jnp.float32)
    out = _run_fused(x_flat, w1t, w2t, bt)
    return out.reshape(n, c, h, w)
